# Initial kernel scaffold; baseline (speedup 1.0000x reference)
import jax
import jax.numpy as jnp
from jax.experimental import pallas as pl


def kernel(x, edge_index, W_rel, W_root):
    raise NotImplementedError("write your pallas kernel here")


# speedup vs baseline: 4.7301x; 4.7301x over previous
import functools

import jax
import jax.numpy as jnp
from jax import lax
from jax.experimental import pallas as pl
from jax.experimental.pallas import tpu as pltpu
from jax.experimental.pallas import tpu_sc as plsc

N_NODES = 10000
D = 128
NC = 2
NS = 16
NW = NC * NS
K = 128
L = 16

SP_ROWS = 10240


def _sc_segment_sum(x, src3, dst3, ch):
    mesh = plsc.VectorSubcoreMesh(core_axis_name="c", subcore_axis_name="s",
                                  num_cores=NC, num_subcores=NS)

    @functools.partial(
        pl.kernel,
        out_type=jax.ShapeDtypeStruct((NC, N_NODES, D), jnp.float32),
        mesh=mesh,
        scratch_types=[
            pltpu.VMEM((ch, K), jnp.int32),
            pltpu.VMEM((ch, K), jnp.int32),
            pltpu.VMEM((K, D), jnp.float32),
            pltpu.VMEM_SHARED((SP_ROWS, D), jnp.float32),
        ],
    )
    def k(x_hbm, src_hbm, dst_hbm, out_hbm, src_v, dst_v, rows_v, acc_sp):
        c = lax.axis_index("c")
        s = lax.axis_index("s")
        wid = c * NS + s

        zeros = jnp.zeros((L,), jnp.float32)

        def zbody(i, _):
            r = i // (D // L)
            col = (i % (D // L)) * L
            rows_v[r, pl.ds(col, L)] = zeros
            return 0

        lax.fori_loop(0, K * (D // L), zbody, 0)
        for j in range(SP_ROWS // (NS * K)):
            pltpu.sync_copy(rows_v, acc_sp.at[pl.ds(s * (SP_ROWS // NS) + j * K, K)])
        plsc.subcore_barrier()

        pltpu.sync_copy(src_hbm.at[wid], src_v)
        pltpu.sync_copy(dst_hbm.at[wid], dst_v)

        def chunk(j, _):
            pltpu.sync_copy(x_hbm.at[src_v.at[j]], rows_v)
            pltpu.sync_copy(rows_v, acc_sp.at[dst_v.at[j]], add=True)
            return 0

        lax.fori_loop(0, ch, chunk, 0)
        plsc.subcore_barrier()

        rpt = (N_NODES // NS) // 8 * 8
        base = s * rpt
        pltpu.sync_copy(acc_sp.at[pl.ds(base, rpt)],
                        out_hbm.at[c, pl.ds(base, rpt)])

        @pl.when(s == NS - 1)
        def _tail():
            t0 = NS * rpt
            pltpu.sync_copy(acc_sp.at[pl.ds(t0, N_NODES - t0)],
                            out_hbm.at[c, pl.ds(t0, N_NODES - t0)])

    return k(x, src3, dst3)


def _tc_combine(partials, x, w_rel_t, w_root_t):
    bn = 2000

    def body(p_ref, x_ref, wr_ref, wo_ref, o_ref):
        agg = p_ref[0] + p_ref[1]
        o_ref[...] = (
            jnp.dot(agg, wr_ref[...], preferred_element_type=jnp.float32)
            + jnp.dot(x_ref[...], wo_ref[...], preferred_element_type=jnp.float32)
        )

    return pl.pallas_call(
        body,
        grid=(N_NODES // bn,),
        in_specs=[
            pl.BlockSpec((NC, bn, D), lambda i: (0, i, 0)),
            pl.BlockSpec((bn, D), lambda i: (i, 0)),
            pl.BlockSpec((D, D), lambda i: (0, 0)),
            pl.BlockSpec((D, D), lambda i: (0, 0)),
        ],
        out_specs=pl.BlockSpec((bn, D), lambda i: (i, 0)),
        out_shape=jax.ShapeDtypeStruct((N_NODES, D), jnp.float32),
    )(partials, x, w_rel_t, w_root_t)


def kernel(x, edge_index, W_rel, W_root):
    e = edge_index.shape[1]
    src = edge_index[0].astype(jnp.int32)
    dst = edge_index[1].astype(jnp.int32)
    e_pad = ((e + NW * K - 1) // (NW * K)) * (NW * K)
    ch = e_pad // (NW * K)
    src = jnp.pad(src, (0, e_pad - e)).reshape(NW, ch, K)
    dst = jnp.pad(dst, (0, e_pad - e), constant_values=N_NODES).reshape(NW, ch, K)

    partials = _sc_segment_sum(x, src, dst, ch)
    return _tc_combine(partials, x, W_rel.T, W_root.T)
